# Initial kernel scaffold; baseline (speedup 1.0000x reference)
import jax
import jax.numpy as jnp
from jax.experimental import pallas as pl


def kernel(x_nchw, w1, b1, w2, b2, gamma, beta):
    raise NotImplementedError("write your pallas kernel here")


# speedup vs baseline: 8.3085x; 8.3085x over previous
import functools

import numpy as np
import jax
import jax.numpy as jnp
from jax.experimental import pallas as pl
from jax.experimental.pallas import tpu as pltpu


def _conv_stats_kernel(x_ref, g_ref, w1_ref, w2_ref, conv_ref, stats_ref):
    v = jnp.maximum(x_ref[0], 0.0)
    p = jnp.dot(v, g_ref[...], preferred_element_type=jnp.float32)
    s = p.shape[1] // 2
    y1 = jnp.dot(w1_ref[...], p[:, :s], preferred_element_type=jnp.float32)
    y2 = jnp.dot(w2_ref[...], p[:, s:], preferred_element_type=jnp.float32)
    y = jnp.concatenate([y1, y2], axis=0)
    conv_ref[0] = y.astype(conv_ref.dtype)
    ycat = jnp.concatenate([y, y * y], axis=0)
    ones = jnp.ones((8, s), jnp.float32)
    stats_ref[0] = jax.lax.dot_general(
        ones, ycat, dimension_numbers=(((1,), (1,)), ((), ())),
        preferred_element_type=jnp.float32)


def _finalize_kernel(stats_ref, gamma_ref, beta_ref,
                     scale_ref, shift_ref, *, count, eps, s):
    tot = jnp.sum(stats_ref[...], axis=0)
    c = gamma_ref.shape[1]
    row = tot[0:1, :]
    inv_n = 1.0 / count
    mean0 = row[:, :c] * inv_n
    var = row[:, c:] * inv_n - mean0 * mean0
    scale = gamma_ref[...] * jax.lax.rsqrt(var + eps)
    shift = beta_ref[...] - mean0 * scale
    scale_ref[...] = jnp.transpose(jnp.broadcast_to(scale, (s, c)))
    shift_ref[...] = jnp.transpose(jnp.broadcast_to(shift, (s, c)))


def _apply_kernel(conv_ref, scale_ref, shift_ref, o_ref):
    y = conv_ref[...].astype(jnp.float32)
    o_ref[...] = y * scale_ref[...][None] + shift_ref[...][None]


def kernel(x_nchw, w1, b1, w2, b2, gamma, beta, *, eps=1e-5):
    n, cin, h, w = x_nchw.shape
    half = w1.shape[0]
    cout = 2 * half
    oh, ow = h // 2, w // 2
    s = oh * ow
    hw = h * w
    rows = n * s

    x_flat = x_nchw.astype(jnp.float32).reshape(n, cin, hw)

    jj = np.arange(s)
    r_, q_ = jj // ow, jj % ow
    g_np = np.zeros((hw, 2 * s), np.float32)
    g_np[(2 * r_) * w + 2 * q_, jj] = 1.0
    g_np[(2 * r_ + 1) * w + (2 * q_ + 1), s + jj] = 1.0
    g = jnp.asarray(g_np)

    w1f = w1.astype(jnp.float32)
    w2f = w2.astype(jnp.float32)
    del b1, b2
    g_row = gamma.astype(jnp.float32).reshape(1, cout)
    beta_row = beta.astype(jnp.float32).reshape(1, cout)

    cparams = pltpu.CompilerParams(
        dimension_semantics=("parallel",),
        vmem_limit_bytes=64 * 1024 * 1024,
    )

    conv, stats = pl.pallas_call(
        _conv_stats_kernel,
        grid=(n,),
        in_specs=[pl.BlockSpec((1, cin, hw), lambda i: (i, 0, 0)),
                  pl.BlockSpec((hw, 2 * s), lambda i: (0, 0)),
                  pl.BlockSpec((half, cin), lambda i: (0, 0)),
                  pl.BlockSpec((half, cin), lambda i: (0, 0))],
        out_specs=(pl.BlockSpec((1, cout, s), lambda i: (i, 0, 0)),
                   pl.BlockSpec((1, 8, 2 * cout), lambda i: (i, 0, 0))),
        out_shape=(jax.ShapeDtypeStruct((n, cout, s), jnp.bfloat16),
                   jax.ShapeDtypeStruct((n, 8, 2 * cout), jnp.float32)),
        compiler_params=cparams,
        cost_estimate=pl.CostEstimate(
            flops=2 * rows * (2 * cin) * cout + 2 * n * cin * hw * 2 * s,
            transcendentals=0,
            bytes_accessed=4 * (n * cin * hw + hw * 2 * s)
            + 2 * n * cout * s + 4 * n * 8 * 2 * cout),
    )(x_flat, g, w1f, w2f)

    scale_t, shift_t = pl.pallas_call(
        functools.partial(_finalize_kernel, count=float(rows), eps=eps, s=s),
        out_shape=(jax.ShapeDtypeStruct((cout, s), jnp.float32),
                   jax.ShapeDtypeStruct((cout, s), jnp.float32)),
    )(stats, g_row, beta_row)

    nb = 4
    while n % nb:
        nb -= 1
    out = pl.pallas_call(
        _apply_kernel,
        grid=(n // nb,),
        in_specs=[pl.BlockSpec((nb, cout, s), lambda i: (i, 0, 0)),
                  pl.BlockSpec((cout, s), lambda i: (0, 0)),
                  pl.BlockSpec((cout, s), lambda i: (0, 0))],
        out_specs=pl.BlockSpec((nb, cout, s), lambda i: (i, 0, 0)),
        out_shape=jax.ShapeDtypeStruct((n, cout, s), jnp.float32),
        compiler_params=cparams,
        cost_estimate=pl.CostEstimate(
            flops=2 * rows * cout,
            transcendentals=0,
            bytes_accessed=2 * n * cout * s + 4 * n * cout * s
            + 8 * cout * s),
    )(conv, scale_t, shift_t)

    return out.reshape(n, cout, oh, ow)
